# baseline (speedup 1.0000x reference)
import functools

import jax
import jax.numpy as jnp
from jax import lax
from jax.experimental import pallas as pl
from jax.experimental.pallas import tpu as pltpu
from jax.experimental.pallas import tpu_sc as plsc

N = 10000
E = 320000
NPAD = 10112
NC = 2
NS = 16
NW = NC * NS
C = 128
H = 64
W = 8
CHW = 160
NCH = NS * CHW
EPAD = NCH * C
DEG_CH = NCH // NW
DEGW = 128

_mesh = plsc.VectorSubcoreMesh(core_axis_name="c", subcore_axis_name="s")


def _make_deg_kernel():
    @functools.partial(
        pl.kernel,
        out_type=jax.ShapeDtypeStruct((NC, NPAD, DEGW), jnp.float32),
        mesh=_mesh,
        scratch_types=[
            pltpu.VMEM((DEG_CH, C), jnp.int32),
            pltpu.VMEM((C, DEGW), jnp.float32),
            pltpu.MemorySpace.VMEM_SHARED((NPAD, DEGW), jnp.float32),
        ],
    )
    def deg_kernel(dst_hbm, ones_hbm, zeros_hbm, out_hbm, dst_v, ones_v, acc):
        c = lax.axis_index("c")
        s = lax.axis_index("s")
        wid = c * NS + s
        rows = NPAD // NS
        pltpu.sync_copy(zeros_hbm.at[pl.ds(s * rows, rows)],
                        acc.at[pl.ds(s * rows, rows)])
        pltpu.sync_copy(dst_hbm.at[pl.ds(wid * DEG_CH, DEG_CH)], dst_v)
        pltpu.sync_copy(ones_hbm, ones_v)
        plsc.subcore_barrier()

        def body(j, carry):
            pltpu.sync_copy(ones_v, acc.at[dst_v.at[j]], add=True)
            return carry

        lax.fori_loop(0, DEG_CH, body, 0)
        plsc.subcore_barrier()

        @pl.when(s == 0)
        def _():
            pltpu.sync_copy(acc, out_hbm.at[c])

    return deg_kernel


def _make_agg_kernel(D):
    @functools.partial(
        pl.kernel,
        out_type=jax.ShapeDtypeStruct((NPAD, D), jnp.float32),
        mesh=_mesh,
        scratch_types=[
            pltpu.VMEM((2, W, C), jnp.int32),
            pltpu.VMEM((2, W, C), jnp.int32),
            pltpu.VMEM((C, D), jnp.float32),
            pltpu.VMEM((C, D), jnp.float32),
            pltpu.MemorySpace.VMEM_SHARED((NPAD, D), jnp.float32),
            pltpu.SemaphoreType.DMA,
            pltpu.SemaphoreType.DMA,
            pltpu.SemaphoreType.DMA,
        ],
    )
    def agg_kernel(g_hbm, src_hbm, dst_hbm, zeros_hbm, out_hbm,
                   srcw, dstw, rows0, rows1, acc, semi, sem0, sem1):
        c = lax.axis_index("c")
        s = lax.axis_index("s")

        rbuf = (rows0, rows1)
        rsem = (sem0, sem1)

        def gather2(idxp, idxk, rb):
            pltpu.async_copy(g_hbm.at[srcw.at[idxp, idxk, pl.ds(0, H)]],
                             rbuf[rb].at[pl.ds(0, H)], rsem[rb])
            pltpu.async_copy(g_hbm.at[srcw.at[idxp, idxk, pl.ds(H, H)]],
                             rbuf[rb].at[pl.ds(H, H)], rsem[rb])

        @pl.when(c == 0)
        def _():
            rows = NPAD // NS
            pltpu.sync_copy(zeros_hbm.at[pl.ds(s * rows, rows)],
                            acc.at[pl.ds(s * rows, rows)])
            plsc.subcore_barrier()

            start = s * CHW
            nw = CHW // W
            pltpu.async_copy(src_hbm.at[pl.ds(start, W)], srcw.at[0], semi)
            pltpu.async_copy(dst_hbm.at[pl.ds(start, W)], dstw.at[0], semi)
            pltpu.make_async_copy(src_hbm.at[pl.ds(0, W)], srcw.at[0], semi).wait()
            pltpu.make_async_copy(src_hbm.at[pl.ds(0, W)], dstw.at[0], semi).wait()
            pltpu.async_copy(src_hbm.at[pl.ds(start + W, W)], srcw.at[1], semi)
            pltpu.async_copy(dst_hbm.at[pl.ds(start + W, W)], dstw.at[1], semi)
            gather2(0, 0, 0)

            def body(w, carry):
                p = jnp.bitwise_and(w, 1)
                base_next2 = start + (w + 2) * W
                for k in range(W):
                    rb = k % 2
                    nb = 1 - rb
                    pltpu.make_async_copy(
                        g_hbm.at[srcw.at[0, 0]], rbuf[rb], rsem[rb]).wait()
                    if k < W - 1:
                        gather2(p, k + 1, nb)
                    pltpu.sync_copy(rbuf[rb], acc.at[dstw.at[p, k]], add=True)
                    if k == W - 1:
                        @pl.when(w + 1 < nw)
                        def _():
                            pltpu.make_async_copy(
                                src_hbm.at[pl.ds(0, W)], srcw.at[0], semi).wait()
                            pltpu.make_async_copy(
                                src_hbm.at[pl.ds(0, W)], dstw.at[0], semi).wait()

                            @pl.when(w + 2 < nw)
                            def _():
                                pltpu.async_copy(
                                    src_hbm.at[pl.ds(base_next2, W)],
                                    srcw.at[p], semi)
                                pltpu.async_copy(
                                    dst_hbm.at[pl.ds(base_next2, W)],
                                    dstw.at[p], semi)

                            gather2(1 - p, 0, 0)
                return carry

            lax.fori_loop(0, nw, body, 0)
            plsc.subcore_barrier()

            @pl.when(s == 0)
            def _():
                pltpu.sync_copy(acc, out_hbm)

    return agg_kernel


_deg_kernel = _make_deg_kernel()
_agg128 = _make_agg_kernel(128)


def _prep_body(x_ref, w_ref, p_ref, g_ref, dinv_ref):
    deg = 1.0 + p_ref[0, :, 0:1] + p_ref[1, :, 0:1]
    dinv = lax.rsqrt(deg)
    h = jnp.dot(x_ref[...], w_ref[...], preferred_element_type=jnp.float32)
    g_ref[...] = dinv * h
    dinv_ref[...] = dinv


def _prep(x_pad, w1, deg_parts):
    return pl.pallas_call(
        _prep_body,
        out_shape=[
            jax.ShapeDtypeStruct((NPAD, 128), jnp.float32),
            jax.ShapeDtypeStruct((NPAD, 1), jnp.float32),
        ],
    )(x_pad, w1, deg_parts)


def _mid_body(s_ref, g_ref, dinv_ref, b_ref, w_ref, out_ref):
    dinv = dinv_ref[...]
    t = dinv * (s_ref[...] + g_ref[...]) + b_ref[...]
    out_ref[...] = dinv * jnp.dot(t, w_ref[...],
                                  preferred_element_type=jnp.float32)


def _mid(s, g, dinv, b, w_next, d_next):
    return pl.pallas_call(
        _mid_body,
        out_shape=jax.ShapeDtypeStruct((NPAD, d_next), jnp.float32),
    )(s, g, dinv, b, w_next)


def _fin_body(s_ref, g_ref, dinv_ref, b_ref, out_ref):
    out_ref[...] = dinv_ref[...] * (s_ref[...] + g_ref[...]) + b_ref[...]


def _fin(s, g, dinv, b):
    return pl.pallas_call(
        _fin_body,
        out_shape=jax.ShapeDtypeStruct((NPAD, 128), jnp.float32),
    )(s, g, dinv, b)


def kernel(x, edge_index, W1, b1, W2, b2, W3, b3):
    src = edge_index[0]
    dst = edge_index[1]
    pad = jnp.full((EPAD - E,), N, dtype=jnp.int32)
    src_p = jnp.concatenate([src, pad]).reshape(NCH, C)
    dst_p = jnp.concatenate([dst, pad]).reshape(NCH, C)

    x_pad = jnp.pad(x, ((0, NPAD - N), (0, 0)))
    z128 = jnp.zeros((NPAD, 128), jnp.float32)
    ones_rows = jnp.ones((C, DEGW), jnp.float32)
    w3_pad = jnp.pad(W3, ((0, 0), (0, 64)))
    b3_pad = jnp.pad(b3, (0, 64))

    deg_parts = _deg_kernel(dst_p, ones_rows, z128)
    g1, dinv = _prep(x_pad, W1, deg_parts)

    s1 = _agg128(g1, src_p, dst_p, z128)
    g2 = _mid(s1, g1, dinv, b1.reshape(1, 128), W2, 128)
    s2 = _agg128(g2, src_p, dst_p, z128)
    g3 = _mid(s2, g2, dinv, b2.reshape(1, 128), w3_pad, 128)
    s3 = _agg128(g3, src_p, dst_p, z128)
    out = _fin(s3, g3, dinv, b3_pad.reshape(1, 128))
    return out[:N, :64]


# speedup vs baseline: 1.0006x; 1.0006x over previous
import functools

import jax
import jax.numpy as jnp
from jax import lax
from jax.experimental import pallas as pl
from jax.experimental.pallas import tpu as pltpu
from jax.experimental.pallas import tpu_sc as plsc

N = 10000
E = 320000
NPAD = 10112
NC = 2
NS = 16
NW = NC * NS
C = 128
H = 64
W = 8
CHW = 160
NCH = NS * CHW
EPAD = NCH * C
DEG_CH = NCH // NW
DEGW = 128

_mesh = plsc.VectorSubcoreMesh(core_axis_name="c", subcore_axis_name="s")


def _make_deg_kernel():
    @functools.partial(
        pl.kernel,
        out_type=jax.ShapeDtypeStruct((NC, NPAD, DEGW), jnp.float32),
        mesh=_mesh,
        scratch_types=[
            pltpu.VMEM((DEG_CH, C), jnp.int32),
            pltpu.VMEM((C, DEGW), jnp.float32),
            pltpu.MemorySpace.VMEM_SHARED((NPAD, DEGW), jnp.float32),
        ],
    )
    def deg_kernel(dst_hbm, ones_hbm, zeros_hbm, out_hbm, dst_v, ones_v, acc):
        c = lax.axis_index("c")
        s = lax.axis_index("s")
        wid = c * NS + s
        rows = NPAD // NS
        pltpu.sync_copy(zeros_hbm.at[pl.ds(s * rows, rows)],
                        acc.at[pl.ds(s * rows, rows)])
        pltpu.sync_copy(dst_hbm.at[pl.ds(wid * DEG_CH, DEG_CH)], dst_v)
        pltpu.sync_copy(ones_hbm, ones_v)
        plsc.subcore_barrier()

        def body(j, carry):
            pltpu.sync_copy(ones_v, acc.at[dst_v.at[j]], add=True)
            return carry

        lax.fori_loop(0, DEG_CH, body, 0)
        plsc.subcore_barrier()

        @pl.when(s == 0)
        def _():
            pltpu.sync_copy(acc, out_hbm.at[c])

    return deg_kernel


def _make_agg_kernel(D):
    @functools.partial(
        pl.kernel,
        out_type=jax.ShapeDtypeStruct((NPAD, D), jnp.float32),
        mesh=_mesh,
        scratch_types=[
            pltpu.VMEM((2, W, C), jnp.int32),
            pltpu.VMEM((2, W, C), jnp.int32),
            pltpu.VMEM((C, D), jnp.float32),
            pltpu.VMEM((C, D), jnp.float32),
            pltpu.MemorySpace.VMEM_SHARED((NPAD, D), jnp.float32),
            pltpu.SemaphoreType.DMA,
            pltpu.SemaphoreType.DMA,
            pltpu.SemaphoreType.DMA,
        ],
    )
    def agg_kernel(g_hbm, src_hbm, dst_hbm, zeros_hbm, out_hbm,
                   srcw, dstw, rows0, rows1, acc, semi, sem0, sem1):
        c = lax.axis_index("c")
        s = lax.axis_index("s")

        rbuf = (rows0, rows1)
        rsem = (sem0, sem1)

        def gather2(idxp, idxk, rb):
            pltpu.async_copy(g_hbm.at[srcw.at[idxp, idxk]],
                             rbuf[rb], rsem[rb])

        @pl.when(c == 0)
        def _():
            rows = NPAD // NS
            pltpu.sync_copy(zeros_hbm.at[pl.ds(s * rows, rows)],
                            acc.at[pl.ds(s * rows, rows)])
            plsc.subcore_barrier()

            start = s * CHW
            nw = CHW // W
            pltpu.async_copy(src_hbm.at[pl.ds(start, W)], srcw.at[0], semi)
            pltpu.async_copy(dst_hbm.at[pl.ds(start, W)], dstw.at[0], semi)
            pltpu.make_async_copy(src_hbm.at[pl.ds(0, W)], srcw.at[0], semi).wait()
            pltpu.make_async_copy(src_hbm.at[pl.ds(0, W)], dstw.at[0], semi).wait()
            pltpu.async_copy(src_hbm.at[pl.ds(start + W, W)], srcw.at[1], semi)
            pltpu.async_copy(dst_hbm.at[pl.ds(start + W, W)], dstw.at[1], semi)
            gather2(0, 0, 0)

            def body(w, carry):
                p = jnp.bitwise_and(w, 1)
                base_next2 = start + (w + 2) * W
                for k in range(W):
                    rb = k % 2
                    nb = 1 - rb
                    pltpu.make_async_copy(
                        g_hbm.at[srcw.at[0, 0]], rbuf[rb], rsem[rb]).wait()
                    if k < W - 1:
                        gather2(p, k + 1, nb)
                    pltpu.sync_copy(rbuf[rb], acc.at[dstw.at[p, k]], add=True)
                    if k == W - 1:
                        @pl.when(w + 1 < nw)
                        def _():
                            pltpu.make_async_copy(
                                src_hbm.at[pl.ds(0, W)], srcw.at[0], semi).wait()
                            pltpu.make_async_copy(
                                src_hbm.at[pl.ds(0, W)], dstw.at[0], semi).wait()

                            @pl.when(w + 2 < nw)
                            def _():
                                pltpu.async_copy(
                                    src_hbm.at[pl.ds(base_next2, W)],
                                    srcw.at[p], semi)
                                pltpu.async_copy(
                                    dst_hbm.at[pl.ds(base_next2, W)],
                                    dstw.at[p], semi)

                            gather2(1 - p, 0, 0)
                return carry

            lax.fori_loop(0, nw, body, 0)
            plsc.subcore_barrier()

            @pl.when(s == 0)
            def _():
                pltpu.sync_copy(acc, out_hbm)

    return agg_kernel


_deg_kernel = _make_deg_kernel()
_agg128 = _make_agg_kernel(128)


def _prep_body(x_ref, w_ref, p_ref, g_ref, dinv_ref):
    deg = 1.0 + p_ref[0, :, 0:1] + p_ref[1, :, 0:1]
    dinv = lax.rsqrt(deg)
    h = jnp.dot(x_ref[...], w_ref[...], preferred_element_type=jnp.float32)
    g_ref[...] = dinv * h
    dinv_ref[...] = dinv


def _prep(x_pad, w1, deg_parts):
    return pl.pallas_call(
        _prep_body,
        out_shape=[
            jax.ShapeDtypeStruct((NPAD, 128), jnp.float32),
            jax.ShapeDtypeStruct((NPAD, 1), jnp.float32),
        ],
    )(x_pad, w1, deg_parts)


def _mid_body(s_ref, g_ref, dinv_ref, b_ref, w_ref, out_ref):
    dinv = dinv_ref[...]
    t = dinv * (s_ref[...] + g_ref[...]) + b_ref[...]
    out_ref[...] = dinv * jnp.dot(t, w_ref[...],
                                  preferred_element_type=jnp.float32)


def _mid(s, g, dinv, b, w_next, d_next):
    return pl.pallas_call(
        _mid_body,
        out_shape=jax.ShapeDtypeStruct((NPAD, d_next), jnp.float32),
    )(s, g, dinv, b, w_next)


def _fin_body(s_ref, g_ref, dinv_ref, b_ref, out_ref):
    out_ref[...] = dinv_ref[...] * (s_ref[...] + g_ref[...]) + b_ref[...]


def _fin(s, g, dinv, b):
    return pl.pallas_call(
        _fin_body,
        out_shape=jax.ShapeDtypeStruct((NPAD, 128), jnp.float32),
    )(s, g, dinv, b)


def kernel(x, edge_index, W1, b1, W2, b2, W3, b3):
    src = edge_index[0]
    dst = edge_index[1]
    pad = jnp.full((EPAD - E,), N, dtype=jnp.int32)
    src_p = jnp.concatenate([src, pad]).reshape(NCH, C)
    dst_p = jnp.concatenate([dst, pad]).reshape(NCH, C)

    x_pad = jnp.pad(x, ((0, NPAD - N), (0, 0)))
    z128 = jnp.zeros((NPAD, 128), jnp.float32)
    ones_rows = jnp.ones((C, DEGW), jnp.float32)
    w3_pad = jnp.pad(W3, ((0, 0), (0, 64)))
    b3_pad = jnp.pad(b3, (0, 64))

    deg_parts = _deg_kernel(dst_p, ones_rows, z128)
    g1, dinv = _prep(x_pad, W1, deg_parts)

    s1 = _agg128(g1, src_p, dst_p, z128)
    g2 = _mid(s1, g1, dinv, b1.reshape(1, 128), W2, 128)
    s2 = _agg128(g2, src_p, dst_p, z128)
    g3 = _mid(s2, g2, dinv, b2.reshape(1, 128), w3_pad, 128)
    s3 = _agg128(g3, src_p, dst_p, z128)
    out = _fin(s3, g3, dinv, b3_pad.reshape(1, 128))
    return out[:N, :64]


# speedup vs baseline: 3.1653x; 3.1633x over previous
import functools

import jax
import jax.numpy as jnp
from jax import lax
from jax.experimental import pallas as pl
from jax.experimental.pallas import tpu as pltpu
from jax.experimental.pallas import tpu_sc as plsc

N = 10000
E = 320000
NPAD = 10112
NC = 2
NS = 16
NW = NC * NS
C = 128
W = 8
CHPW = 80
NCH = NW * CHPW
EPAD = NCH * C
DEG_CH = NCH // NW
DEGW = 128

_mesh = plsc.VectorSubcoreMesh(core_axis_name="c", subcore_axis_name="s")


def _make_deg_kernel():
    @functools.partial(
        pl.kernel,
        out_type=jax.ShapeDtypeStruct((NC, NPAD, DEGW), jnp.float32),
        mesh=_mesh,
        scratch_types=[
            pltpu.VMEM((DEG_CH, C), jnp.int32),
            pltpu.VMEM((C, DEGW), jnp.float32),
            pltpu.MemorySpace.VMEM_SHARED((NPAD, DEGW), jnp.float32),
        ],
    )
    def deg_kernel(dst_hbm, ones_hbm, zeros_hbm, out_hbm, dst_v, ones_v, acc):
        c = lax.axis_index("c")
        s = lax.axis_index("s")
        wid = c * NS + s
        rows = NPAD // NS
        pltpu.sync_copy(zeros_hbm.at[pl.ds(s * rows, rows)],
                        acc.at[pl.ds(s * rows, rows)])
        pltpu.sync_copy(dst_hbm.at[pl.ds(wid * DEG_CH, DEG_CH)], dst_v)
        pltpu.sync_copy(ones_hbm, ones_v)
        plsc.subcore_barrier()

        def body(j, carry):
            pltpu.sync_copy(ones_v, acc.at[dst_v.at[j]], add=True)
            return carry

        lax.fori_loop(0, DEG_CH, body, 0)
        plsc.subcore_barrier()

        @pl.when(s == 0)
        def _():
            pltpu.sync_copy(acc, out_hbm.at[c])

    return deg_kernel


def _make_agg_kernel(D):
    @functools.partial(
        pl.kernel,
        out_type=jax.ShapeDtypeStruct((NC, NPAD, D), jnp.float32),
        mesh=_mesh,
        scratch_types=[
            pltpu.VMEM((2, W, C), jnp.int32),
            pltpu.VMEM((2, W, C), jnp.int32),
            pltpu.VMEM((C, D), jnp.float32),
            pltpu.VMEM((C, D), jnp.float32),
            pltpu.MemorySpace.VMEM_SHARED((NPAD, D), jnp.float32),
            pltpu.SemaphoreType.DMA,
            pltpu.SemaphoreType.DMA,
            pltpu.SemaphoreType.DMA,
        ],
    )
    def agg_kernel(g_hbm, src_hbm, dst_hbm, zeros_hbm, out_hbm,
                   srcw, dstw, rows0, rows1, acc, semi, sem0, sem1):
        c = lax.axis_index("c")
        s = lax.axis_index("s")

        rbuf = (rows0, rows1)
        rsem = (sem0, sem1)

        def gather2(idxp, idxk, rb):
            pltpu.async_copy(g_hbm.at[srcw.at[idxp, idxk]],
                             rbuf[rb], rsem[rb])

        if True:
            rows = NPAD // NS
            pltpu.sync_copy(zeros_hbm.at[pl.ds(s * rows, rows)],
                            acc.at[pl.ds(s * rows, rows)])
            plsc.subcore_barrier()

            wid = c * NS + s
            start = wid * CHPW
            nw = CHPW // W
            pltpu.async_copy(src_hbm.at[pl.ds(start, W)], srcw.at[0], semi)
            pltpu.async_copy(dst_hbm.at[pl.ds(start, W)], dstw.at[0], semi)
            pltpu.make_async_copy(src_hbm.at[pl.ds(0, W)], srcw.at[0], semi).wait()
            pltpu.make_async_copy(src_hbm.at[pl.ds(0, W)], dstw.at[0], semi).wait()
            pltpu.async_copy(src_hbm.at[pl.ds(start + W, W)], srcw.at[1], semi)
            pltpu.async_copy(dst_hbm.at[pl.ds(start + W, W)], dstw.at[1], semi)
            gather2(0, 0, 0)

            def body(w, carry):
                p = jnp.bitwise_and(w, 1)
                base_next2 = start + (w + 2) * W
                for k in range(W):
                    rb = k % 2
                    nb = 1 - rb
                    pltpu.make_async_copy(
                        g_hbm.at[srcw.at[0, 0]], rbuf[rb], rsem[rb]).wait()
                    if k < W - 1:
                        gather2(p, k + 1, nb)
                    pltpu.sync_copy(rbuf[rb], acc.at[dstw.at[p, k]], add=True)
                    if k == W - 1:
                        @pl.when(w + 1 < nw)
                        def _():
                            pltpu.make_async_copy(
                                src_hbm.at[pl.ds(0, W)], srcw.at[0], semi).wait()
                            pltpu.make_async_copy(
                                src_hbm.at[pl.ds(0, W)], dstw.at[0], semi).wait()

                            @pl.when(w + 2 < nw)
                            def _():
                                pltpu.async_copy(
                                    src_hbm.at[pl.ds(base_next2, W)],
                                    srcw.at[p], semi)
                                pltpu.async_copy(
                                    dst_hbm.at[pl.ds(base_next2, W)],
                                    dstw.at[p], semi)

                            gather2(1 - p, 0, 0)
                return carry

            lax.fori_loop(0, nw, body, 0)
            plsc.subcore_barrier()

            @pl.when(s == 0)
            def _():
                pltpu.sync_copy(acc, out_hbm.at[c])

    return agg_kernel


_deg_kernel = _make_deg_kernel()
_agg128 = _make_agg_kernel(128)


def _prep_body(x_ref, w_ref, p_ref, g_ref, dinv_ref):
    deg = 1.0 + p_ref[0, :, 0:1] + p_ref[1, :, 0:1]
    dinv = lax.rsqrt(deg)
    h = jnp.dot(x_ref[...], w_ref[...], preferred_element_type=jnp.float32)
    g_ref[...] = dinv * h
    dinv_ref[...] = dinv


def _prep(x_pad, w1, deg_parts):
    return pl.pallas_call(
        _prep_body,
        out_shape=[
            jax.ShapeDtypeStruct((NPAD, 128), jnp.float32),
            jax.ShapeDtypeStruct((NPAD, 1), jnp.float32),
        ],
    )(x_pad, w1, deg_parts)


def _mid_body(s_ref, g_ref, dinv_ref, b_ref, w_ref, out_ref):
    dinv = dinv_ref[...]
    t = dinv * (s_ref[0] + s_ref[1] + g_ref[...]) + b_ref[...]
    out_ref[...] = dinv * jnp.dot(t, w_ref[...],
                                  preferred_element_type=jnp.float32)


def _mid(s, g, dinv, b, w_next, d_next):
    return pl.pallas_call(
        _mid_body,
        out_shape=jax.ShapeDtypeStruct((NPAD, d_next), jnp.float32),
    )(s, g, dinv, b, w_next)


def _fin_body(s_ref, g_ref, dinv_ref, b_ref, out_ref):
    out_ref[...] = dinv_ref[...] * (s_ref[0] + s_ref[1] + g_ref[...]) + b_ref[...]


def _fin(s, g, dinv, b):
    return pl.pallas_call(
        _fin_body,
        out_shape=jax.ShapeDtypeStruct((NPAD, 128), jnp.float32),
    )(s, g, dinv, b)


def kernel(x, edge_index, W1, b1, W2, b2, W3, b3):
    src = edge_index[0]
    dst = edge_index[1]
    pad = N + (jnp.arange(EPAD - E, dtype=jnp.int32) % (NPAD - N))
    src_p = jnp.concatenate([src, pad]).reshape(NCH, C)
    dst_p = jnp.concatenate([dst, pad]).reshape(NCH, C)

    x_pad = jnp.pad(x, ((0, NPAD - N), (0, 0)))
    z128 = jnp.zeros((NPAD, 128), jnp.float32)
    ones_rows = jnp.ones((C, DEGW), jnp.float32)
    w3_pad = jnp.pad(W3, ((0, 0), (0, 64)))
    b3_pad = jnp.pad(b3, (0, 64))

    deg_parts = _deg_kernel(dst_p, ones_rows, z128)
    g1, dinv = _prep(x_pad, W1, deg_parts)

    s1 = _agg128(g1, src_p, dst_p, z128)
    g2 = _mid(s1, g1, dinv, b1.reshape(1, 128), W2, 128)
    s2 = _agg128(g2, src_p, dst_p, z128)
    g3 = _mid(s2, g2, dinv, b2.reshape(1, 128), w3_pad, 128)
    s3 = _agg128(g3, src_p, dst_p, z128)
    out = _fin(s3, g3, dinv, b3_pad.reshape(1, 128))
    return out[:N, :64]


# speedup vs baseline: 3.1785x; 1.0042x over previous
import functools

import jax
import jax.numpy as jnp
from jax import lax
from jax.experimental import pallas as pl
from jax.experimental.pallas import tpu as pltpu
from jax.experimental.pallas import tpu_sc as plsc

N = 10000
E = 320000
NPAD = 10112
NC = 2
NS = 16
NW = NC * NS
C = 128
W = 8
CHPW = 80
NCH = NW * CHPW
EPAD = NCH * C
DEG_CH = NCH // NW
DEGW = 128

_mesh = plsc.VectorSubcoreMesh(core_axis_name="c", subcore_axis_name="s")


def _make_deg_kernel():
    @functools.partial(
        pl.kernel,
        out_type=jax.ShapeDtypeStruct((NC, NPAD, DEGW), jnp.float32),
        mesh=_mesh,
        scratch_types=[
            pltpu.VMEM((DEG_CH, C), jnp.int32),
            pltpu.VMEM((C, DEGW), jnp.float32),
            pltpu.MemorySpace.VMEM_SHARED((NPAD, DEGW), jnp.float32),
        ],
    )
    def deg_kernel(dst_hbm, ones_hbm, zeros_hbm, out_hbm, dst_v, ones_v, acc):
        c = lax.axis_index("c")
        s = lax.axis_index("s")
        wid = c * NS + s
        rows = NPAD // NS
        pltpu.sync_copy(zeros_hbm.at[pl.ds(s * rows, rows)],
                        acc.at[pl.ds(s * rows, rows)])
        pltpu.sync_copy(dst_hbm.at[pl.ds(wid * DEG_CH, DEG_CH)], dst_v)
        pltpu.sync_copy(ones_hbm, ones_v)
        plsc.subcore_barrier()

        def body(j, carry):
            pltpu.sync_copy(ones_v, acc.at[dst_v.at[j]], add=True)
            return carry

        lax.fori_loop(0, DEG_CH, body, 0)
        plsc.subcore_barrier()

        @pl.when(s == 0)
        def _():
            pltpu.sync_copy(acc, out_hbm.at[c])

    return deg_kernel


def _make_agg_kernel(D):
    @functools.partial(
        pl.kernel,
        out_type=jax.ShapeDtypeStruct((NC, NPAD, D), jnp.float32),
        mesh=_mesh,
        scratch_types=[
            pltpu.VMEM((2, W, C), jnp.int32),
            pltpu.VMEM((2, W, C), jnp.int32),
            pltpu.VMEM((C, D), jnp.float32),
            pltpu.VMEM((C, D), jnp.float32),
            pltpu.MemorySpace.VMEM_SHARED((NPAD, D), jnp.float32),
            pltpu.SemaphoreType.DMA,
            pltpu.SemaphoreType.DMA,
            pltpu.SemaphoreType.DMA,
        ],
    )
    def agg_kernel(g_hbm, src_hbm, dst_hbm, zeros_hbm, out_hbm,
                   srcw, dstw, rows0, rows1, acc, semi, sem0, sem1):
        c = lax.axis_index("c")
        s = lax.axis_index("s")

        rbuf = (rows0, rows1)
        rsem = (sem0, sem1)

        def gather2(idxp, idxk, rb):
            pltpu.async_copy(g_hbm.at[srcw.at[idxp, idxk, pl.ds(0, 64)]],
                             rbuf[rb].at[pl.ds(0, 64)], rsem[rb])
            pltpu.async_copy(g_hbm.at[srcw.at[idxp, idxk, pl.ds(64, 64)]],
                             rbuf[rb].at[pl.ds(64, 64)], rsem[rb])

        if True:
            rows = NPAD // NS
            pltpu.sync_copy(zeros_hbm.at[pl.ds(s * rows, rows)],
                            acc.at[pl.ds(s * rows, rows)])
            plsc.subcore_barrier()

            wid = c * NS + s
            start = wid * CHPW
            nw = CHPW // W
            pltpu.async_copy(src_hbm.at[pl.ds(start, W)], srcw.at[0], semi)
            pltpu.async_copy(dst_hbm.at[pl.ds(start, W)], dstw.at[0], semi)
            pltpu.make_async_copy(src_hbm.at[pl.ds(0, W)], srcw.at[0], semi).wait()
            pltpu.make_async_copy(src_hbm.at[pl.ds(0, W)], dstw.at[0], semi).wait()
            pltpu.async_copy(src_hbm.at[pl.ds(start + W, W)], srcw.at[1], semi)
            pltpu.async_copy(dst_hbm.at[pl.ds(start + W, W)], dstw.at[1], semi)
            gather2(0, 0, 0)

            def body(w, carry):
                p = jnp.bitwise_and(w, 1)
                base_next2 = start + (w + 2) * W
                for k in range(W):
                    rb = k % 2
                    nb = 1 - rb
                    pltpu.make_async_copy(
                        g_hbm.at[srcw.at[0, 0]], rbuf[rb], rsem[rb]).wait()
                    if k < W - 1:
                        gather2(p, k + 1, nb)
                    pltpu.sync_copy(rbuf[rb], acc.at[dstw.at[p, k]], add=True)
                    if k == W - 1:
                        @pl.when(w + 1 < nw)
                        def _():
                            pltpu.make_async_copy(
                                src_hbm.at[pl.ds(0, W)], srcw.at[0], semi).wait()
                            pltpu.make_async_copy(
                                src_hbm.at[pl.ds(0, W)], dstw.at[0], semi).wait()

                            @pl.when(w + 2 < nw)
                            def _():
                                pltpu.async_copy(
                                    src_hbm.at[pl.ds(base_next2, W)],
                                    srcw.at[p], semi)
                                pltpu.async_copy(
                                    dst_hbm.at[pl.ds(base_next2, W)],
                                    dstw.at[p], semi)

                            gather2(1 - p, 0, 0)
                return carry

            lax.fori_loop(0, nw, body, 0)
            plsc.subcore_barrier()

            @pl.when(s == 0)
            def _():
                pltpu.sync_copy(acc, out_hbm.at[c])

    return agg_kernel


_deg_kernel = _make_deg_kernel()
_agg128 = _make_agg_kernel(128)


def _prep_body(x_ref, w_ref, p_ref, g_ref, dinv_ref):
    deg = 1.0 + p_ref[0, :, 0:1] + p_ref[1, :, 0:1]
    dinv = lax.rsqrt(deg)
    h = jnp.dot(x_ref[...], w_ref[...], preferred_element_type=jnp.float32)
    g_ref[...] = dinv * h
    dinv_ref[...] = dinv


def _prep(x_pad, w1, deg_parts):
    return pl.pallas_call(
        _prep_body,
        out_shape=[
            jax.ShapeDtypeStruct((NPAD, 128), jnp.float32),
            jax.ShapeDtypeStruct((NPAD, 1), jnp.float32),
        ],
    )(x_pad, w1, deg_parts)


def _mid_body(s_ref, g_ref, dinv_ref, b_ref, w_ref, out_ref):
    dinv = dinv_ref[...]
    t = dinv * (s_ref[0] + s_ref[1] + g_ref[...]) + b_ref[...]
    out_ref[...] = dinv * jnp.dot(t, w_ref[...],
                                  preferred_element_type=jnp.float32)


def _mid(s, g, dinv, b, w_next, d_next):
    return pl.pallas_call(
        _mid_body,
        out_shape=jax.ShapeDtypeStruct((NPAD, d_next), jnp.float32),
    )(s, g, dinv, b, w_next)


def _fin_body(s_ref, g_ref, dinv_ref, b_ref, out_ref):
    out_ref[...] = dinv_ref[...] * (s_ref[0] + s_ref[1] + g_ref[...]) + b_ref[...]


def _fin(s, g, dinv, b):
    return pl.pallas_call(
        _fin_body,
        out_shape=jax.ShapeDtypeStruct((NPAD, 128), jnp.float32),
    )(s, g, dinv, b)


def kernel(x, edge_index, W1, b1, W2, b2, W3, b3):
    src = edge_index[0]
    dst = edge_index[1]
    pad = N + (jnp.arange(EPAD - E, dtype=jnp.int32) % (NPAD - N))
    src_p = jnp.concatenate([src, pad]).reshape(NCH, C)
    dst_p = jnp.concatenate([dst, pad]).reshape(NCH, C)

    x_pad = jnp.pad(x, ((0, NPAD - N), (0, 0)))
    z128 = jnp.zeros((NPAD, 128), jnp.float32)
    ones_rows = jnp.ones((C, DEGW), jnp.float32)
    w3_pad = jnp.pad(W3, ((0, 0), (0, 64)))
    b3_pad = jnp.pad(b3, (0, 64))

    deg_parts = _deg_kernel(dst_p, ones_rows, z128)
    g1, dinv = _prep(x_pad, W1, deg_parts)

    s1 = _agg128(g1, src_p, dst_p, z128)
    g2 = _mid(s1, g1, dinv, b1.reshape(1, 128), W2, 128)
    s2 = _agg128(g2, src_p, dst_p, z128)
    g3 = _mid(s2, g2, dinv, b2.reshape(1, 128), w3_pad, 128)
    s3 = _agg128(g3, src_p, dst_p, z128)
    out = _fin(s3, g3, dinv, b3_pad.reshape(1, 128))
    return out[:N, :64]


# speedup vs baseline: 3.1821x; 1.0011x over previous
import functools

import jax
import jax.numpy as jnp
from jax import lax
from jax.experimental import pallas as pl
from jax.experimental.pallas import tpu as pltpu
from jax.experimental.pallas import tpu_sc as plsc

N = 10000
E = 320000
NPAD = 10112
NC = 2
NS = 16
NW = NC * NS
C = 128
W = 8
CHPW = 80
NCH = NW * CHPW
EPAD = NCH * C
DEG_CH = NCH // NW
DEGW = 128

_mesh = plsc.VectorSubcoreMesh(core_axis_name="c", subcore_axis_name="s")


def _make_deg_kernel():
    @functools.partial(
        pl.kernel,
        out_type=jax.ShapeDtypeStruct((NC, NPAD, DEGW), jnp.float32),
        mesh=_mesh,
        scratch_types=[
            pltpu.VMEM((DEG_CH, C), jnp.int32),
            pltpu.VMEM((C, DEGW), jnp.float32),
            pltpu.MemorySpace.VMEM_SHARED((NPAD, DEGW), jnp.float32),
        ],
    )
    def deg_kernel(dst_hbm, ones_hbm, zeros_hbm, out_hbm, dst_v, ones_v, acc):
        c = lax.axis_index("c")
        s = lax.axis_index("s")
        wid = c * NS + s
        rows = NPAD // NS
        pltpu.sync_copy(zeros_hbm.at[pl.ds(s * rows, rows)],
                        acc.at[pl.ds(s * rows, rows)])
        pltpu.sync_copy(dst_hbm.at[pl.ds(wid * DEG_CH, DEG_CH)], dst_v)
        pltpu.sync_copy(ones_hbm, ones_v)
        plsc.subcore_barrier()

        def body(j, carry):
            pltpu.sync_copy(ones_v, acc.at[dst_v.at[j]], add=True)
            return carry

        lax.fori_loop(0, DEG_CH, body, 0)
        plsc.subcore_barrier()

        @pl.when(s == 0)
        def _():
            pltpu.sync_copy(acc, out_hbm.at[c])

    return deg_kernel


def _make_agg_kernel(D):
    @functools.partial(
        pl.kernel,
        out_type=jax.ShapeDtypeStruct((NC, NPAD, D), jnp.float32),
        mesh=_mesh,
        scratch_types=[
            pltpu.VMEM((2, W, C), jnp.int32),
            pltpu.VMEM((2, W, C), jnp.int32),
            pltpu.VMEM((C, D), jnp.float32),
            pltpu.VMEM((C, D), jnp.float32),
            pltpu.MemorySpace.VMEM_SHARED((NPAD, D), jnp.float32),
            pltpu.SemaphoreType.DMA,
            pltpu.SemaphoreType.DMA,
            pltpu.SemaphoreType.DMA,
        ],
    )
    def agg_kernel(g_hbm, src_hbm, dst_hbm, zeros_hbm, out_hbm,
                   srcw, dstw, rows0, rows1, acc, semi, sem0, sem1):
        c = lax.axis_index("c")
        s = lax.axis_index("s")

        rbuf = (rows0, rows1)
        rsem = (sem0, sem1)

        def gather2(idxp, idxk, rb):
            pltpu.async_copy(g_hbm.at[srcw.at[idxp, idxk, pl.ds(0, 64)]],
                             rbuf[rb].at[pl.ds(0, 64)], rsem[rb])
            pltpu.async_copy(g_hbm.at[srcw.at[idxp, idxk, pl.ds(64, 64)]],
                             rbuf[rb].at[pl.ds(64, 64)], rsem[rb])

        if True:
            rows = NPAD // NS
            pltpu.sync_copy(zeros_hbm.at[pl.ds(s * rows, rows)],
                            acc.at[pl.ds(s * rows, rows)])
            plsc.subcore_barrier()

            wid = c * NS + s
            start = wid * CHPW
            nw = CHPW // W
            pltpu.async_copy(src_hbm.at[pl.ds(start, W)], srcw.at[0], semi)
            pltpu.async_copy(dst_hbm.at[pl.ds(start, W)], dstw.at[0], semi)
            pltpu.make_async_copy(src_hbm.at[pl.ds(0, W)], srcw.at[0], semi).wait()
            pltpu.make_async_copy(src_hbm.at[pl.ds(0, W)], dstw.at[0], semi).wait()
            pltpu.async_copy(src_hbm.at[pl.ds(start + W, W)], srcw.at[1], semi)
            pltpu.async_copy(dst_hbm.at[pl.ds(start + W, W)], dstw.at[1], semi)
            gather2(0, 0, 0)

            def body(w, carry):
                p = jnp.bitwise_and(w, 1)
                base_next2 = start + (w + 2) * W
                for k in range(W):
                    rb = k % 2
                    nb = 1 - rb
                    pltpu.make_async_copy(
                        g_hbm.at[srcw.at[0, 0]], rbuf[rb], rsem[rb]).wait()
                    if k < W - 1:
                        gather2(p, k + 1, nb)
                    pltpu.sync_copy(rbuf[rb], acc.at[dstw.at[p, k]], add=True)
                    if k == W - 1:
                        @pl.when(w + 1 < nw)
                        def _():
                            pltpu.make_async_copy(
                                src_hbm.at[pl.ds(0, W)], srcw.at[0], semi).wait()
                            pltpu.make_async_copy(
                                src_hbm.at[pl.ds(0, W)], dstw.at[0], semi).wait()

                            @pl.when(w + 2 < nw)
                            def _():
                                pltpu.async_copy(
                                    src_hbm.at[pl.ds(base_next2, W)],
                                    srcw.at[p], semi)
                                pltpu.async_copy(
                                    dst_hbm.at[pl.ds(base_next2, W)],
                                    dstw.at[p], semi)

                            gather2(1 - p, 0, 0)
                return carry

            lax.fori_loop(0, nw, body, 0)
            plsc.subcore_barrier()

            @pl.when(s == 0)
            def _():
                pltpu.sync_copy(acc, out_hbm.at[c])

    return agg_kernel


_deg_kernel = _make_deg_kernel()
_agg128 = _make_agg_kernel(128)


def _prep_body(x_ref, w_ref, p_ref, g_ref, dinv_ref):
    deg = 1.0 + p_ref[0, 0:N, 0:1] + p_ref[1, 0:N, 0:1]
    dinv = lax.rsqrt(deg)
    h = jnp.dot(x_ref[...], w_ref[...], preferred_element_type=jnp.float32)
    g_ref[...] = dinv * h
    dinv_ref[...] = dinv


def _prep(x, w1, deg_parts):
    return pl.pallas_call(
        _prep_body,
        out_shape=[
            jax.ShapeDtypeStruct((N, 128), jnp.float32),
            jax.ShapeDtypeStruct((N, 1), jnp.float32),
        ],
    )(x, w1, deg_parts)


def _mid_body(s_ref, g_ref, dinv_ref, b_ref, w_ref, out_ref):
    dinv = dinv_ref[...]
    t = dinv * (s_ref[0, 0:N] + s_ref[1, 0:N] + g_ref[...]) + b_ref[...]
    out_ref[...] = dinv * jnp.dot(t, w_ref[...],
                                  preferred_element_type=jnp.float32)


def _mid(s, g, dinv, b, w_next, d_next):
    return pl.pallas_call(
        _mid_body,
        out_shape=jax.ShapeDtypeStruct((N, d_next), jnp.float32),
    )(s, g, dinv, b, w_next)


def _fin_body(s_ref, g_ref, dinv_ref, b_ref, out_ref):
    t = dinv_ref[...] * (s_ref[0, 0:N] + s_ref[1, 0:N] + g_ref[...]) + b_ref[...]
    out_ref[...] = t[:, 0:64]


def _fin(s, g, dinv, b):
    return pl.pallas_call(
        _fin_body,
        out_shape=jax.ShapeDtypeStruct((N, 64), jnp.float32),
    )(s, g, dinv, b)


def kernel(x, edge_index, W1, b1, W2, b2, W3, b3):
    src = edge_index[0]
    dst = edge_index[1]
    npad_extra = EPAD - E
    pad_src = jnp.arange(npad_extra, dtype=jnp.int32) % N
    pad_dst = N + (jnp.arange(npad_extra, dtype=jnp.int32) % (NPAD - N))
    src_p = jnp.concatenate([src, pad_src]).reshape(NCH, C)
    dst_p = jnp.concatenate([dst, pad_dst]).reshape(NCH, C)

    z128 = jnp.zeros((NPAD, 128), jnp.float32)
    ones_rows = jnp.ones((C, DEGW), jnp.float32)
    w3_pad = jnp.pad(W3, ((0, 0), (0, 64)))
    b3_pad = jnp.pad(b3, (0, 64))

    deg_parts = _deg_kernel(dst_p, ones_rows, z128)
    g1, dinv = _prep(x, W1, deg_parts)

    s1 = _agg128(g1, src_p, dst_p, z128)
    g2 = _mid(s1, g1, dinv, b1.reshape(1, 128), W2, 128)
    s2 = _agg128(g2, src_p, dst_p, z128)
    g3 = _mid(s2, g2, dinv, b2.reshape(1, 128), w3_pad, 128)
    s3 = _agg128(g3, src_p, dst_p, z128)
    return _fin(s3, g3, dinv, b3_pad.reshape(1, 64 + 64))


# speedup vs baseline: 3.2394x; 1.0180x over previous
import functools

import jax
import jax.numpy as jnp
from jax import lax
from jax.experimental import pallas as pl
from jax.experimental.pallas import tpu as pltpu
from jax.experimental.pallas import tpu_sc as plsc

N = 10000
E = 320000
NPAD = 10112
NC = 2
NS = 16
NW = NC * NS
C = 128
W = 16
CHPW = 80
NCH = NW * CHPW
EPAD = NCH * C
DEG_CH = NCH // NW
DEGW = 128

_mesh = plsc.VectorSubcoreMesh(core_axis_name="c", subcore_axis_name="s")


def _make_deg_kernel():
    @functools.partial(
        pl.kernel,
        out_type=jax.ShapeDtypeStruct((NC, NPAD, DEGW), jnp.float32),
        mesh=_mesh,
        scratch_types=[
            pltpu.VMEM((DEG_CH, C), jnp.int32),
            pltpu.VMEM((C, DEGW), jnp.float32),
            pltpu.MemorySpace.VMEM_SHARED((NPAD, DEGW), jnp.float32),
        ],
    )
    def deg_kernel(dst_hbm, ones_hbm, zeros_hbm, out_hbm, dst_v, ones_v, acc):
        c = lax.axis_index("c")
        s = lax.axis_index("s")
        wid = c * NS + s
        rows = NPAD // NS
        pltpu.sync_copy(zeros_hbm.at[pl.ds(s * rows, rows)],
                        acc.at[pl.ds(s * rows, rows)])
        pltpu.sync_copy(dst_hbm.at[pl.ds(wid * DEG_CH, DEG_CH)], dst_v)
        pltpu.sync_copy(ones_hbm, ones_v)
        plsc.subcore_barrier()

        def body(j, carry):
            pltpu.sync_copy(ones_v, acc.at[dst_v.at[j]], add=True)
            return carry

        lax.fori_loop(0, DEG_CH, body, 0)
        plsc.subcore_barrier()

        @pl.when(s == 0)
        def _():
            pltpu.sync_copy(acc, out_hbm.at[c])

    return deg_kernel


def _make_agg_kernel(D):
    @functools.partial(
        pl.kernel,
        out_type=jax.ShapeDtypeStruct((NC, NPAD, D), jnp.float32),
        mesh=_mesh,
        scratch_types=[
            pltpu.VMEM((2, W, C), jnp.int32),
            pltpu.VMEM((2, W, C), jnp.int32),
            pltpu.VMEM((C, D), jnp.float32),
            pltpu.VMEM((C, D), jnp.float32),
            pltpu.MemorySpace.VMEM_SHARED((NPAD, D), jnp.float32),
            pltpu.SemaphoreType.DMA,
            pltpu.SemaphoreType.DMA,
            pltpu.SemaphoreType.DMA,
        ],
    )
    def agg_kernel(g_hbm, src_hbm, dst_hbm, zeros_hbm, out_hbm,
                   srcw, dstw, rows0, rows1, acc, semi, sem0, sem1):
        c = lax.axis_index("c")
        s = lax.axis_index("s")

        rbuf = (rows0, rows1)
        rsem = (sem0, sem1)

        def gather2(idxp, idxk, rb):
            pltpu.async_copy(g_hbm.at[srcw.at[idxp, idxk, pl.ds(0, 64)]],
                             rbuf[rb].at[pl.ds(0, 64)], rsem[rb])
            pltpu.async_copy(g_hbm.at[srcw.at[idxp, idxk, pl.ds(64, 64)]],
                             rbuf[rb].at[pl.ds(64, 64)], rsem[rb])

        if True:
            rows = NPAD // NS
            pltpu.sync_copy(zeros_hbm.at[pl.ds(s * rows, rows)],
                            acc.at[pl.ds(s * rows, rows)])
            plsc.subcore_barrier()

            wid = c * NS + s
            start = wid * CHPW
            nw = CHPW // W
            pltpu.async_copy(src_hbm.at[pl.ds(start, W)], srcw.at[0], semi)
            pltpu.async_copy(dst_hbm.at[pl.ds(start, W)], dstw.at[0], semi)
            pltpu.make_async_copy(src_hbm.at[pl.ds(0, W)], srcw.at[0], semi).wait()
            pltpu.make_async_copy(src_hbm.at[pl.ds(0, W)], dstw.at[0], semi).wait()
            pltpu.async_copy(src_hbm.at[pl.ds(start + W, W)], srcw.at[1], semi)
            pltpu.async_copy(dst_hbm.at[pl.ds(start + W, W)], dstw.at[1], semi)
            gather2(0, 0, 0)

            def body(w, carry):
                p = jnp.bitwise_and(w, 1)
                base_next2 = start + (w + 2) * W
                for k in range(W):
                    rb = k % 2
                    nb = 1 - rb
                    pltpu.make_async_copy(
                        g_hbm.at[srcw.at[0, 0]], rbuf[rb], rsem[rb]).wait()
                    if k < W - 1:
                        gather2(p, k + 1, nb)
                    pltpu.sync_copy(rbuf[rb], acc.at[dstw.at[p, k]], add=True)
                    if k == W - 1:
                        @pl.when(w + 1 < nw)
                        def _():
                            pltpu.make_async_copy(
                                src_hbm.at[pl.ds(0, W)], srcw.at[0], semi).wait()
                            pltpu.make_async_copy(
                                src_hbm.at[pl.ds(0, W)], dstw.at[0], semi).wait()

                            @pl.when(w + 2 < nw)
                            def _():
                                pltpu.async_copy(
                                    src_hbm.at[pl.ds(base_next2, W)],
                                    srcw.at[p], semi)
                                pltpu.async_copy(
                                    dst_hbm.at[pl.ds(base_next2, W)],
                                    dstw.at[p], semi)

                            gather2(1 - p, 0, 0)
                return carry

            lax.fori_loop(0, nw, body, 0)
            plsc.subcore_barrier()

            @pl.when(s == 0)
            def _():
                pltpu.sync_copy(acc, out_hbm.at[c])

    return agg_kernel


_deg_kernel = _make_deg_kernel()
_agg128 = _make_agg_kernel(128)


def _prep_body(x_ref, w_ref, p_ref, g_ref, dinv_ref):
    deg = 1.0 + p_ref[0, 0:N, 0:1] + p_ref[1, 0:N, 0:1]
    dinv = lax.rsqrt(deg)
    h = jnp.dot(x_ref[...], w_ref[...], preferred_element_type=jnp.float32)
    g_ref[...] = dinv * h
    dinv_ref[...] = dinv


def _prep(x, w1, deg_parts):
    return pl.pallas_call(
        _prep_body,
        out_shape=[
            jax.ShapeDtypeStruct((N, 128), jnp.float32),
            jax.ShapeDtypeStruct((N, 1), jnp.float32),
        ],
    )(x, w1, deg_parts)


def _mid_body(s_ref, g_ref, dinv_ref, b_ref, w_ref, out_ref):
    dinv = dinv_ref[...]
    t = dinv * (s_ref[0, 0:N] + s_ref[1, 0:N] + g_ref[...]) + b_ref[...]
    out_ref[...] = dinv * jnp.dot(t, w_ref[...],
                                  preferred_element_type=jnp.float32)


def _mid(s, g, dinv, b, w_next, d_next):
    return pl.pallas_call(
        _mid_body,
        out_shape=jax.ShapeDtypeStruct((N, d_next), jnp.float32),
    )(s, g, dinv, b, w_next)


def _fin_body(s_ref, g_ref, dinv_ref, b_ref, out_ref):
    t = dinv_ref[...] * (s_ref[0, 0:N] + s_ref[1, 0:N] + g_ref[...]) + b_ref[...]
    out_ref[...] = t[:, 0:64]


def _fin(s, g, dinv, b):
    return pl.pallas_call(
        _fin_body,
        out_shape=jax.ShapeDtypeStruct((N, 64), jnp.float32),
    )(s, g, dinv, b)


def kernel(x, edge_index, W1, b1, W2, b2, W3, b3):
    src = edge_index[0]
    dst = edge_index[1]
    npad_extra = EPAD - E
    pad_src = jnp.arange(npad_extra, dtype=jnp.int32) % N
    pad_dst = N + (jnp.arange(npad_extra, dtype=jnp.int32) % (NPAD - N))
    src_p = jnp.concatenate([src, pad_src]).reshape(NCH, C)
    dst_p = jnp.concatenate([dst, pad_dst]).reshape(NCH, C)

    z128 = jnp.zeros((NPAD, 128), jnp.float32)
    ones_rows = jnp.ones((C, DEGW), jnp.float32)
    w3_pad = jnp.pad(W3, ((0, 0), (0, 64)))
    b3_pad = jnp.pad(b3, (0, 64))

    deg_parts = _deg_kernel(dst_p, ones_rows, z128)
    g1, dinv = _prep(x, W1, deg_parts)

    s1 = _agg128(g1, src_p, dst_p, z128)
    g2 = _mid(s1, g1, dinv, b1.reshape(1, 128), W2, 128)
    s2 = _agg128(g2, src_p, dst_p, z128)
    g3 = _mid(s2, g2, dinv, b2.reshape(1, 128), w3_pad, 128)
    s3 = _agg128(g3, src_p, dst_p, z128)
    return _fin(s3, g3, dinv, b3_pad.reshape(1, 64 + 64))
